# speedup vs baseline: 2.7424x; 2.7424x over previous
import jax
import jax.numpy as jnp
from jax.experimental import pallas as pl
from jax.experimental.pallas import tpu as pltpu

_TOP_K = 2
_EXPERTS = 16
_BLK = 1024


def _router_block_kernel(x_ref, wt_ref, b_ref, out_ref, idx_ref):
    x = x_ref[...]
    wt = wt_ref[...]
    logits = jnp.dot(x, wt, preferred_element_type=jnp.float32) + b_ref[...]
    lane = jax.lax.broadcasted_iota(jnp.int32, logits.shape, 1)

    v1 = jnp.max(logits, axis=1, keepdims=True)
    idx1 = jnp.min(jnp.where(logits >= v1, lane, _EXPERTS), axis=1, keepdims=True)
    masked = jnp.where(lane == idx1, -jnp.inf, logits)
    v2 = jnp.max(masked, axis=1, keepdims=True)
    idx2 = jnp.min(jnp.where(masked >= v2, lane, _EXPERTS), axis=1, keepdims=True)

    e = jnp.exp(v2 - v1)
    denom = 1.0 + e
    p1 = 1.0 / denom
    p2 = e / denom
    out_ref[...] = (jnp.where(lane == idx1, p1, 0.0)
                    + jnp.where(lane == idx2, p2, 0.0))
    idx_ref[...] = jnp.concatenate([idx1, idx2], axis=1)


def kernel(x, route_W, route_b, noise_W, noise_b):
    del noise_W, noise_b
    tokens = x.shape[0]
    wt = route_W.T
    b2d = route_b.reshape(1, _EXPERTS)
    grid = (tokens // _BLK,)
    out, idx = pl.pallas_call(
        _router_block_kernel,
        grid=grid,
        in_specs=[
            pl.BlockSpec((_BLK, x.shape[1]), lambda i: (i, 0)),
            pl.BlockSpec((x.shape[1], _EXPERTS), lambda i: (0, 0)),
            pl.BlockSpec((1, _EXPERTS), lambda i: (0, 0)),
        ],
        out_specs=[
            pl.BlockSpec((_BLK, _EXPERTS), lambda i: (i, 0)),
            pl.BlockSpec((_BLK, _TOP_K), lambda i: (i, 0)),
        ],
        out_shape=[
            jax.ShapeDtypeStruct((tokens, _EXPERTS), jnp.float32),
            jax.ShapeDtypeStruct((tokens, _TOP_K), jnp.int32),
        ],
        compiler_params=pltpu.CompilerParams(
            dimension_semantics=("parallel",),
        ),
    )(x, wt, b2d)
    return (out, idx)
